# baseline (speedup 1.0000x reference)
import functools

import jax
import jax.numpy as jnp
from jax import lax
from jax.experimental import pallas as pl
from jax.experimental.pallas import tpu as pltpu
from jax.experimental.pallas import tpu_sc as plsc

N_NODES = 10000
D = 128
N_EDGES = 320000

NC = 2
NS = 16
NW = NC * NS
EPW = N_EDGES // NW
K = 50
CPW = EPW // K
N_PAD = 10240
ROWS_PT = N_PAD // NS

_MESH = plsc.VectorSubcoreMesh(core_axis_name="c", subcore_axis_name="s")


HR = N_PAD // D
EPW_PAD = N_PAD


def _deg_body(dst_hbm, zer_hbm, out_hbm, table_v, dst_v):
  c = lax.axis_index("c")
  s = lax.axis_index("s")
  w = c * NS + s
  pltpu.sync_copy(zer_hbm, table_v)
  pltpu.sync_copy(dst_hbm.at[w], dst_v)
  ones = jnp.full((16,), 1.0, dtype=jnp.float32)

  def body(r, carry):
    for g in range(D // 16):
      d = dst_v[r, pl.ds(g * 16, 16)]
      hi = lax.shift_right_logical(d, 7)
      lo = lax.bitwise_and(d, 127)
      plsc.addupdate_scatter(table_v, [hi, lo], ones)
    return carry

  lax.fori_loop(0, HR, body, 0)
  pltpu.sync_copy(table_v, out_hbm.at[w])


_deg_kernel = pl.kernel(
    _deg_body,
    out_type=jax.ShapeDtypeStruct((NW, HR, D), jnp.float32),
    mesh=_MESH,
    scratch_types=[
        pltpu.VMEM((HR, D), jnp.float32),
        pltpu.VMEM((HR, D), jnp.int32),
    ],
    compiler_params=pltpu.CompilerParams(needs_layout_passes=False),
)


def _dinv_reduce_body(degp_ref, out_ref):
  deg = jnp.sum(degp_ref[...], axis=0) + 1.0
  out_ref[...] = lax.rsqrt(deg)


_dinv_reduce = pl.pallas_call(
    _dinv_reduce_body,
    out_shape=jax.ShapeDtypeStruct((HR, D), jnp.float32),
)


NB = 2
NH = 2
HALF = CPW // NH
_TAIL = HALF - (HALF // NB) * NB


def _scatter_chunk(g_hbm, acc, src_v, dst_v, bufs, sems, j, b):
  pltpu.make_async_copy(g_hbm.at[src_v.at[j]], bufs[b], sems[b]).wait()

  @pl.when(j + NB < HALF)
  def _refill():
    pltpu.async_copy(g_hbm.at[src_v.at[j + NB]], bufs[b], sems[b])

  pltpu.sync_copy(bufs[b], acc.at[dst_v.at[j]], add=True)


def _scatter_body(g_hbm, src_hbm, dst_hbm, zer_hbm, out_hbm,
                  acc, src_v, dst_v, r0, r1, s0, s1):
  bufs = (r0, r1)
  sems = (s0, s1)
  c = lax.axis_index("c")
  s = lax.axis_index("s")
  w = c * NS + s
  pltpu.sync_copy(zer_hbm, r0)
  ZR = 40
  for z in range(ROWS_PT // ZR):
    pltpu.sync_copy(r0.at[pl.ds(0, ZR)],
                    acc.at[pl.ds(s * ROWS_PT + z * ZR, ZR)])
  plsc.subcore_barrier()

  for h in range(NH):
    pltpu.sync_copy(src_hbm.at[w, h], src_v)
    pltpu.sync_copy(dst_hbm.at[w, h], dst_v)
    for b in range(NB):
      pltpu.async_copy(g_hbm.at[src_v.at[b]], bufs[b], sems[b])

    def body(i, carry):
      for b in range(NB):
        _scatter_chunk(g_hbm, acc, src_v, dst_v, bufs, sems, i * NB + b, b)
      return carry

    lax.fori_loop(0, HALF // NB, body, 0)
    for t in range(_TAIL):
      j = (HALF // NB) * NB + t
      _scatter_chunk(g_hbm, acc, src_v, dst_v, bufs, sems,
                     jnp.int32(j), j % NB)

  plsc.subcore_barrier()
  pltpu.sync_copy(acc.at[pl.ds(s * ROWS_PT, ROWS_PT)],
                  out_hbm.at[c, pl.ds(s * ROWS_PT, ROWS_PT)])


_scatter_kernel = pl.kernel(
    _scatter_body,
    out_type=jax.ShapeDtypeStruct((NC, N_PAD, D), jnp.float32),
    mesh=_MESH,
    scratch_types=[
        pltpu.VMEM_SHARED((N_PAD, D), jnp.float32),
        pltpu.VMEM((HALF, K), jnp.int32),
        pltpu.VMEM((HALF, K), jnp.int32),
        pltpu.VMEM((K, D), jnp.float32),
        pltpu.VMEM((K, D), jnp.float32),
        pltpu.SemaphoreType.DMA,
        pltpu.SemaphoreType.DMA,
    ],
)


ROWS_TC = 1000


def _dense1_body(x_ref, w_ref, dinv_ref, g_ref):
  h = jnp.dot(x_ref[...], w_ref[...], preferred_element_type=jnp.float32)
  g_ref[...] = h * dinv_ref[...]


def _dense2_body(s_ref, g_ref, dinv_ref, b_ref, w_ref, out_ref):
  dinv = dinv_ref[...]
  pre = dinv * (s_ref[0] + s_ref[1] + g_ref[...]) + b_ref[...]
  h = jnp.maximum(pre, 0.0)
  out_ref[...] = jnp.dot(h, w_ref[...],
                         preferred_element_type=jnp.float32) * dinv


def _dense3_body(s_ref, g_ref, dinv_ref, b_ref, wfc_ref, bfc_ref, out_ref):
  h = dinv_ref[...] * (s_ref[0] + s_ref[1] + g_ref[...]) + b_ref[...]
  out_ref[...] = jnp.dot(h, wfc_ref[...],
                         preferred_element_type=jnp.float32) + bfc_ref[...]


_GRID = (N_NODES // ROWS_TC,)
_ROWBLK = pl.BlockSpec((ROWS_TC, D), lambda i: (i, 0))
_DINVBLK = pl.BlockSpec((ROWS_TC, 1), lambda i: (i, 0))
_SBLK = pl.BlockSpec((NC, ROWS_TC, D), lambda i: (0, i, 0))
_WBLK = pl.BlockSpec((D, D), lambda i: (0, 0))
_BBLK = pl.BlockSpec((1, D), lambda i: (0, 0))

_dense1 = pl.pallas_call(
    _dense1_body,
    grid=_GRID,
    in_specs=[_ROWBLK, _WBLK, _DINVBLK],
    out_specs=_ROWBLK,
    out_shape=jax.ShapeDtypeStruct((N_NODES, D), jnp.float32),
)

_dense2 = pl.pallas_call(
    _dense2_body,
    grid=_GRID,
    in_specs=[_SBLK, _ROWBLK, _DINVBLK, _BBLK, _WBLK],
    out_specs=_ROWBLK,
    out_shape=jax.ShapeDtypeStruct((N_NODES, D), jnp.float32),
)

_dense3 = pl.pallas_call(
    _dense3_body,
    grid=_GRID,
    in_specs=[_SBLK, _ROWBLK, _DINVBLK, _BBLK,
              pl.BlockSpec((D, 1), lambda i: (0, 0)),
              pl.BlockSpec((1, 1), lambda i: (0, 0))],
    out_specs=pl.BlockSpec((ROWS_TC, 1), lambda i: (i, 0)),
    out_shape=jax.ShapeDtypeStruct((N_NODES, 1), jnp.float32),
)


def kernel(x, edge_index, W1, b1, W2, b2, Wfc, bfc):
  src = edge_index[0].astype(jnp.int32).reshape(NW, NH, HALF, K)
  dst = edge_index[1].astype(jnp.int32).reshape(NW, NH, HALF, K)
  dst_pad = jnp.concatenate(
      [dst.reshape(N_EDGES),
       jnp.full((NW * EPW_PAD - N_EDGES,), N_PAD - 1, jnp.int32)]
  ).reshape(NW, HR, D)
  zer = jnp.zeros((HR, D), jnp.float32)
  zerk = jnp.zeros((K, D), jnp.float32)
  b1r = b1.reshape(1, D)
  b2r = b2.reshape(1, D)
  bfcr = bfc.reshape(1, 1)

  degp = _deg_kernel(dst_pad, zer)
  dinv = _dinv_reduce(degp).reshape(N_PAD, 1)
  g1 = _dense1(x, W1, dinv)
  s1 = _scatter_kernel(g1, src, dst, zerk)
  g2 = _dense2(s1, g1, dinv, b1r, W2)
  s2 = _scatter_kernel(g2, src, dst, zerk)
  return _dense3(s2, g2, dinv, b2r, Wfc, bfcr)


# speedup vs baseline: 1.1603x; 1.1603x over previous
import functools

import jax
import jax.numpy as jnp
from jax import lax
from jax.experimental import pallas as pl
from jax.experimental.pallas import tpu as pltpu
from jax.experimental.pallas import tpu_sc as plsc

N_NODES = 10000
D = 128
N_EDGES = 320000

NC = 2
NS = 16
NW = NC * NS
EPW = N_EDGES // NW
K = 100
CPW = EPW // K
N_PAD = 10240
ROWS_PT = N_PAD // NS

_MESH = plsc.VectorSubcoreMesh(core_axis_name="c", subcore_axis_name="s")


HR = N_PAD // D
EPW_PAD = N_PAD


def _deg_body(dst_hbm, zer_hbm, out_hbm, table_v, dst_v):
  c = lax.axis_index("c")
  s = lax.axis_index("s")
  w = c * NS + s
  pltpu.sync_copy(zer_hbm, table_v)
  pltpu.sync_copy(dst_hbm.at[w], dst_v)
  ones = jnp.full((16,), 1.0, dtype=jnp.float32)

  def body(r, carry):
    for g in range(D // 16):
      d = dst_v[r, pl.ds(g * 16, 16)]
      hi = lax.shift_right_logical(d, 7)
      lo = lax.bitwise_and(d, 127)
      plsc.addupdate_scatter(table_v, [hi, lo], ones)
    return carry

  lax.fori_loop(0, HR, body, 0)
  pltpu.sync_copy(table_v, out_hbm.at[w])


_deg_kernel = pl.kernel(
    _deg_body,
    out_type=jax.ShapeDtypeStruct((NW, HR, D), jnp.float32),
    mesh=_MESH,
    scratch_types=[
        pltpu.VMEM((HR, D), jnp.float32),
        pltpu.VMEM((HR, D), jnp.int32),
    ],
    compiler_params=pltpu.CompilerParams(needs_layout_passes=False),
)


def _dinv_reduce_body(degp_ref, out_ref):
  deg = jnp.sum(degp_ref[...], axis=0) + 1.0
  out_ref[...] = lax.rsqrt(deg)


_dinv_reduce = pl.pallas_call(
    _dinv_reduce_body,
    out_shape=jax.ShapeDtypeStruct((HR, D), jnp.float32),
)


NB = 2
NH = 2
HALF = CPW // NH
_TAIL = HALF - (HALF // NB) * NB


def _scatter_chunk(g_hbm, acc, src_v, dst_v, bufs, sems, j, b):
  pltpu.make_async_copy(g_hbm.at[src_v.at[j]], bufs[b], sems[b]).wait()

  @pl.when(j + NB < HALF)
  def _refill():
    pltpu.async_copy(g_hbm.at[src_v.at[j + NB]], bufs[b], sems[b])

  pltpu.sync_copy(bufs[b], acc.at[dst_v.at[j]], add=True)


def _scatter_body(g_hbm, src_hbm, dst_hbm, zer_hbm, out_hbm,
                  acc, src_v, dst_v, r0, r1, s0, s1):
  bufs = (r0, r1)
  sems = (s0, s1)
  c = lax.axis_index("c")
  s = lax.axis_index("s")
  w = c * NS + s
  pltpu.sync_copy(zer_hbm, r0)
  ZR = 40
  for z in range(ROWS_PT // ZR):
    pltpu.sync_copy(r0.at[pl.ds(0, ZR)],
                    acc.at[pl.ds(s * ROWS_PT + z * ZR, ZR)])
  plsc.subcore_barrier()

  for h in range(NH):
    pltpu.sync_copy(src_hbm.at[w, h], src_v)
    pltpu.sync_copy(dst_hbm.at[w, h], dst_v)
    for b in range(NB):
      pltpu.async_copy(g_hbm.at[src_v.at[b]], bufs[b], sems[b])

    def body(i, carry):
      for b in range(NB):
        _scatter_chunk(g_hbm, acc, src_v, dst_v, bufs, sems, i * NB + b, b)
      return carry

    lax.fori_loop(0, HALF // NB, body, 0)
    for t in range(_TAIL):
      j = (HALF // NB) * NB + t
      _scatter_chunk(g_hbm, acc, src_v, dst_v, bufs, sems,
                     jnp.int32(j), j % NB)

  plsc.subcore_barrier()
  pltpu.sync_copy(acc.at[pl.ds(s * ROWS_PT, ROWS_PT)],
                  out_hbm.at[c, pl.ds(s * ROWS_PT, ROWS_PT)])


_scatter_kernel = pl.kernel(
    _scatter_body,
    out_type=jax.ShapeDtypeStruct((NC, N_PAD, D), jnp.float32),
    mesh=_MESH,
    scratch_types=[
        pltpu.VMEM_SHARED((N_PAD, D), jnp.float32),
        pltpu.VMEM((HALF, K), jnp.int32),
        pltpu.VMEM((HALF, K), jnp.int32),
        pltpu.VMEM((K, D), jnp.float32),
        pltpu.VMEM((K, D), jnp.float32),
        pltpu.SemaphoreType.DMA,
        pltpu.SemaphoreType.DMA,
    ],
)


ROWS_TC = 1000


def _dense1_body(x_ref, w_ref, dinv_ref, g_ref):
  h = jnp.dot(x_ref[...], w_ref[...], preferred_element_type=jnp.float32)
  g_ref[...] = h * dinv_ref[...]


def _dense2_body(s_ref, g_ref, dinv_ref, b_ref, w_ref, out_ref):
  dinv = dinv_ref[...]
  pre = dinv * (s_ref[0] + s_ref[1] + g_ref[...]) + b_ref[...]
  h = jnp.maximum(pre, 0.0)
  out_ref[...] = jnp.dot(h, w_ref[...],
                         preferred_element_type=jnp.float32) * dinv


def _dense3_body(s_ref, g_ref, dinv_ref, b_ref, wfc_ref, bfc_ref, out_ref):
  h = dinv_ref[...] * (s_ref[0] + s_ref[1] + g_ref[...]) + b_ref[...]
  out_ref[...] = jnp.dot(h, wfc_ref[...],
                         preferred_element_type=jnp.float32) + bfc_ref[...]


_GRID = (N_NODES // ROWS_TC,)
_ROWBLK = pl.BlockSpec((ROWS_TC, D), lambda i: (i, 0))
_DINVBLK = pl.BlockSpec((ROWS_TC, 1), lambda i: (i, 0))
_SBLK = pl.BlockSpec((NC, ROWS_TC, D), lambda i: (0, i, 0))
_WBLK = pl.BlockSpec((D, D), lambda i: (0, 0))
_BBLK = pl.BlockSpec((1, D), lambda i: (0, 0))

_dense1 = pl.pallas_call(
    _dense1_body,
    grid=_GRID,
    in_specs=[_ROWBLK, _WBLK, _DINVBLK],
    out_specs=_ROWBLK,
    out_shape=jax.ShapeDtypeStruct((N_NODES, D), jnp.float32),
)

_dense2 = pl.pallas_call(
    _dense2_body,
    grid=_GRID,
    in_specs=[_SBLK, _ROWBLK, _DINVBLK, _BBLK, _WBLK],
    out_specs=_ROWBLK,
    out_shape=jax.ShapeDtypeStruct((N_NODES, D), jnp.float32),
)

_dense3 = pl.pallas_call(
    _dense3_body,
    grid=_GRID,
    in_specs=[_SBLK, _ROWBLK, _DINVBLK, _BBLK,
              pl.BlockSpec((D, 1), lambda i: (0, 0)),
              pl.BlockSpec((1, 1), lambda i: (0, 0))],
    out_specs=pl.BlockSpec((ROWS_TC, 1), lambda i: (i, 0)),
    out_shape=jax.ShapeDtypeStruct((N_NODES, 1), jnp.float32),
)


def kernel(x, edge_index, W1, b1, W2, b2, Wfc, bfc):
  src = edge_index[0].astype(jnp.int32).reshape(NW, NH, HALF, K)
  dst = edge_index[1].astype(jnp.int32).reshape(NW, NH, HALF, K)
  dst_pad = jnp.concatenate(
      [dst.reshape(N_EDGES),
       jnp.full((NW * EPW_PAD - N_EDGES,), N_PAD - 1, jnp.int32)]
  ).reshape(NW, HR, D)
  zer = jnp.zeros((HR, D), jnp.float32)
  zerk = jnp.zeros((K, D), jnp.float32)
  b1r = b1.reshape(1, D)
  b2r = b2.reshape(1, D)
  bfcr = bfc.reshape(1, 1)

  degp = _deg_kernel(dst_pad, zer)
  dinv = _dinv_reduce(degp).reshape(N_PAD, 1)
  g1 = _dense1(x, W1, dinv)
  s1 = _scatter_kernel(g1, src, dst, zerk)
  g2 = _dense2(s1, g1, dinv, b1r, W2)
  s2 = _scatter_kernel(g2, src, dst, zerk)
  return _dense3(s2, g2, dinv, b2r, Wfc, bfcr)
